# baseline (speedup 1.0000x reference)
import functools

import jax
import jax.numpy as jnp
from jax import lax
from jax.experimental import pallas as pl
from jax.experimental.pallas import tpu as pltpu
from jax.experimental.pallas import tpu_sc as plsc

B = 100
N = 200
FLAT = N * N
NUM_EL = int(0.3 * N * N)
H1 = 512
H2 = 1024
OUT = 2
EPS = 1e-5

KBLK = 2048
KT = (FLAT + KBLK - 1) // KBLK

NC = 2
NS = 16
L = 16
NW = NC * NS
ROUNDS = (B + NW - 1) // NW
NBINS = 256
HIST_W = NBINS * L


def _sc_thresh_body(adj_ref, thr_ref, buf, hist, stage, sem):
    wid = lax.axis_index("s") * NC + lax.axis_index("c")
    lane = lax.iota(jnp.int32, L)
    lane_base = lane * NBINS
    ones = jnp.ones((L,), jnp.int32)

    for r in range(ROUNDS):
        g = r * NW + wid

        @pl.when(g < B)
        def _():
            pltpu.sync_copy(adj_ref.at[g], buf)

            def run_pass(shift, prefix, target, first):
                @plsc.parallel_loop(0, HIST_W // L, unroll=8)
                def _(i):
                    hist[pl.ds(i * L, L)] = jnp.zeros((L,), jnp.int32)

                @plsc.parallel_loop(0, FLAT // L, unroll=8)
                def _(i):
                    v = buf[pl.ds(i * L, L)]
                    bits = lax.bitcast_convert_type(jnp.abs(v), jnp.int32)
                    digit = lax.shift_right_logical(bits, shift) & 0xFF
                    idx = digit + lane_base
                    if first:
                        plsc.addupdate_scatter(hist, [idx], ones)
                    else:
                        m = lax.shift_right_logical(bits, shift + 8) == prefix
                        plsc.addupdate_scatter(hist, [idx], ones, mask=m)

                def srch(j, carry):
                    cnt, d_star, excl = carry

                    def mrg(l, a):
                        return a + hist[pl.ds(l * NBINS + j * L, L)]

                    v = lax.fori_loop(0, L, mrg, jnp.zeros((L,), jnp.int32))
                    cs = plsc.cumsum(v)
                    tot = jnp.sum(v)
                    cross = (cnt <= target) & (cnt + tot > target)
                    inb = (cnt + cs) <= target
                    d_loc = jnp.sum(inb.astype(jnp.int32))
                    excl_loc = cnt + jnp.max(jnp.where(inb, cs, 0))
                    d_star = jnp.where(cross, j * L + d_loc, d_star)
                    excl = jnp.where(cross, excl_loc, excl)
                    return (cnt + tot, d_star, excl)

                _, d_star, excl = lax.fori_loop(
                    0, NBINS // L, srch,
                    (jnp.int32(0), jnp.int32(0), jnp.int32(0)))
                return (prefix << 8) | d_star, target - excl

            target = jnp.int32(FLAT - NUM_EL)
            prefix = jnp.int32(0)
            prefix, target = run_pass(23, prefix, target, True)
            prefix, target = run_pass(15, prefix, target, False)
            prefix, target = run_pass(7, prefix, target, False)

            thr_bits = prefix << 7
            thr = lax.bitcast_convert_type(
                jnp.broadcast_to(thr_bits, (L,)), jnp.float32)
            stage[...] = thr
            pltpu.sync_copy(stage, thr_ref.at[g])


_sc_thresh = functools.partial(
    pl.kernel,
    out_type=jax.ShapeDtypeStruct((B, L), jnp.float32),
    mesh=plsc.VectorSubcoreMesh(core_axis_name="c", subcore_axis_name="s",
                                num_cores=NC, num_subcores=NS),
    compiler_params=pltpu.CompilerParams(needs_layout_passes=False),
    scratch_types=[
        pltpu.VMEM((FLAT,), jnp.float32),
        pltpu.VMEM((HIST_W,), jnp.int32),
        pltpu.VMEM((L,), jnp.float32),
        pltpu.SemaphoreType.DMA,
    ],
)(_sc_thresh_body)


def _mlp_body(x_ref, adj_ref, thr_ref, w1_ref, b1_ref, g1_ref, be1_ref,
              w2_ref, b2_ref, g2_ref, be2_ref, w3_ref, b3_ref,
              out_ref, acc_ref):
    i = pl.program_id(0)

    @pl.when(i == 0)
    def _():
        acc_ref[...] = jnp.zeros_like(acc_ref)

    thr = thr_ref[:, 0:1]
    col = i * KBLK + jax.lax.broadcasted_iota(jnp.int32, (B, KBLK), 1)
    keep = (col < FLAT) & (jnp.abs(adj_ref[...]) >= thr)
    xm = jnp.where(keep, x_ref[...], 0.0)
    row = i * KBLK + jax.lax.broadcasted_iota(jnp.int32, (KBLK, H1), 0)
    w1 = jnp.where(row < FLAT, w1_ref[...], 0.0)
    acc_ref[...] += jnp.dot(xm, w1, preferred_element_type=jnp.float32,
                            precision=jax.lax.Precision.DEFAULT)

    @pl.when(i == KT - 1)
    def _():
        s = 1.0 / (1.0 + EPS) ** 0.5
        h = jnp.maximum(acc_ref[...] + b1_ref[...], 0.0)
        h = g1_ref[...] * h * s + be1_ref[...]
        h = jnp.maximum(jnp.dot(h, w2_ref[...], preferred_element_type=jnp.float32)
                        + b2_ref[...], 0.0)
        h = g2_ref[...] * h * s + be2_ref[...]
        lg = jnp.dot(h, w3_ref[...], preferred_element_type=jnp.float32) + b3_ref[...]
        c = jax.lax.broadcasted_iota(jnp.int32, lg.shape, 1)
        neg = jnp.where(c < OUT, lg, -jnp.inf)
        m = jnp.max(neg, axis=1, keepdims=True)
        ex = jnp.where(c < OUT, jnp.exp(lg - m), 0.0)
        lse = m + jnp.log(jnp.sum(ex, axis=1, keepdims=True))
        out_ref[...] = lg - lse


def kernel(x, adj_logits, W1, b1, gamma1, beta1, W2, b2, gamma2, beta2, W3, b3):
    adj = adj_logits.reshape(B, FLAT)

    thr = _sc_thresh(adj)

    w3p = jnp.pad(W3, ((0, 0), (0, 128 - OUT)))
    b3p = jnp.pad(b3, (0, 128 - OUT)).reshape(1, 128)

    out = pl.pallas_call(
        _mlp_body,
        grid=(KT,),
        in_specs=[
            pl.BlockSpec((B, KBLK), lambda i: (0, i)),
            pl.BlockSpec((B, KBLK), lambda i: (0, i)),
            pl.BlockSpec((B, L), lambda i: (0, 0)),
            pl.BlockSpec((KBLK, H1), lambda i: (i, 0)),
            pl.BlockSpec((1, H1), lambda i: (0, 0)),
            pl.BlockSpec((1, H1), lambda i: (0, 0)),
            pl.BlockSpec((1, H1), lambda i: (0, 0)),
            pl.BlockSpec((H1, H2), lambda i: (0, 0)),
            pl.BlockSpec((1, H2), lambda i: (0, 0)),
            pl.BlockSpec((1, H2), lambda i: (0, 0)),
            pl.BlockSpec((1, H2), lambda i: (0, 0)),
            pl.BlockSpec((H2, 128), lambda i: (0, 0)),
            pl.BlockSpec((1, 128), lambda i: (0, 0)),
        ],
        out_specs=pl.BlockSpec((B, 128), lambda i: (0, 0)),
        out_shape=jax.ShapeDtypeStruct((B, 128), jnp.float32),
        scratch_shapes=[pltpu.VMEM((B, H1), jnp.float32)],
    )(x, adj, thr, W1, b1.reshape(1, H1), gamma1.reshape(1, H1),
      beta1.reshape(1, H1), W2, b2.reshape(1, H2), gamma2.reshape(1, H2),
      beta2.reshape(1, H2), w3p, b3p)

    return out[:, :OUT]


# speedup vs baseline: 1.0295x; 1.0295x over previous
import functools

import jax
import jax.numpy as jnp
from jax import lax
from jax.experimental import pallas as pl
from jax.experimental.pallas import tpu as pltpu
from jax.experimental.pallas import tpu_sc as plsc

B = 100
N = 200
FLAT = N * N
NUM_EL = int(0.3 * N * N)
H1 = 512
H2 = 1024
OUT = 2
EPS = 1e-5

KBLK = 2048
KT = (FLAT + KBLK - 1) // KBLK

NC = 2
NS = 16
L = 16
NW = NC * NS
ROUNDS = (B + NW - 1) // NW
NBINS = 256
HIST_W = NBINS * L


def _sc_thresh_body(adj_ref, thr_ref, buf0, buf1, hist, stage, sem0, sem1):
    wid = lax.axis_index("s") * NC + lax.axis_index("c")
    lane = lax.iota(jnp.int32, L)
    lane_base = lane * NBINS
    ones = jnp.ones((L,), jnp.int32)
    bufs = (buf0, buf1)
    sems = (sem0, sem1)

    def start_fetch(r):
        g = r * NW + wid

        @pl.when(g < B)
        def _():
            pltpu.make_async_copy(adj_ref.at[g], bufs[r % 2], sems[r % 2]).start()

    start_fetch(0)
    for r in range(ROUNDS):
        g = r * NW + wid
        buf = bufs[r % 2]

        @pl.when(g < B)
        def _():
            pltpu.make_async_copy(adj_ref.at[g], buf, sems[r % 2]).wait()

        if r + 1 < ROUNDS:
            start_fetch(r + 1)

        @pl.when(g < B)
        def _():

            def run_pass(shift, prefix, target, first):
                @plsc.parallel_loop(0, HIST_W // L, unroll=8)
                def _(i):
                    hist[pl.ds(i * L, L)] = jnp.zeros((L,), jnp.int32)

                @plsc.parallel_loop(0, FLAT // L, unroll=16)
                def _(i):
                    v = buf[pl.ds(i * L, L)]
                    bits = lax.bitcast_convert_type(jnp.abs(v), jnp.int32)
                    digit = lax.shift_right_logical(bits, shift) & 0xFF
                    idx = digit + lane_base
                    if first:
                        plsc.addupdate_scatter(hist, [idx], ones)
                    else:
                        m = lax.shift_right_logical(bits, shift + 8) == prefix
                        plsc.addupdate_scatter(hist, [idx], ones, mask=m)

                def srch(j, carry):
                    cnt, d_star, excl = carry

                    def mrg(l, a):
                        return a + hist[pl.ds(l * NBINS + j * L, L)]

                    v = lax.fori_loop(0, L, mrg, jnp.zeros((L,), jnp.int32))
                    cs = plsc.cumsum(v)
                    tot = jnp.sum(v)
                    cross = (cnt <= target) & (cnt + tot > target)
                    inb = (cnt + cs) <= target
                    d_loc = jnp.sum(inb.astype(jnp.int32))
                    excl_loc = cnt + jnp.max(jnp.where(inb, cs, 0))
                    d_star = jnp.where(cross, j * L + d_loc, d_star)
                    excl = jnp.where(cross, excl_loc, excl)
                    return (cnt + tot, d_star, excl)

                _, d_star, excl = lax.fori_loop(
                    0, NBINS // L, srch,
                    (jnp.int32(0), jnp.int32(0), jnp.int32(0)))
                return (prefix << 8) | d_star, target - excl

            target = jnp.int32(FLAT - NUM_EL)
            prefix = jnp.int32(0)
            prefix, target = run_pass(23, prefix, target, True)
            prefix, target = run_pass(15, prefix, target, False)
            prefix, target = run_pass(7, prefix, target, False)

            thr_bits = prefix << 7
            thr = lax.bitcast_convert_type(
                jnp.broadcast_to(thr_bits, (L,)), jnp.float32)
            stage[...] = thr
            pltpu.sync_copy(stage, thr_ref.at[g])


_sc_thresh = functools.partial(
    pl.kernel,
    out_type=jax.ShapeDtypeStruct((B, L), jnp.float32),
    mesh=plsc.VectorSubcoreMesh(core_axis_name="c", subcore_axis_name="s",
                                num_cores=NC, num_subcores=NS),
    compiler_params=pltpu.CompilerParams(needs_layout_passes=False),
    scratch_types=[
        pltpu.VMEM((FLAT,), jnp.float32),
        pltpu.VMEM((FLAT,), jnp.float32),
        pltpu.VMEM((HIST_W,), jnp.int32),
        pltpu.VMEM((L,), jnp.float32),
        pltpu.SemaphoreType.DMA,
        pltpu.SemaphoreType.DMA,
    ],
)(_sc_thresh_body)


def _mlp_body(x_ref, adj_ref, thr_ref, w1_ref, b1_ref, g1_ref, be1_ref,
              w2_ref, b2_ref, g2_ref, be2_ref, w3_ref, b3_ref,
              out_ref, acc_ref):
    i = pl.program_id(0)

    @pl.when(i == 0)
    def _():
        acc_ref[...] = jnp.zeros_like(acc_ref)

    thr = thr_ref[:, 0:1]
    col = i * KBLK + jax.lax.broadcasted_iota(jnp.int32, (B, KBLK), 1)
    keep = (col < FLAT) & (jnp.abs(adj_ref[...]) >= thr)
    xm = jnp.where(keep, x_ref[...], 0.0)
    row = i * KBLK + jax.lax.broadcasted_iota(jnp.int32, (KBLK, H1), 0)
    w1 = jnp.where(row < FLAT, w1_ref[...], 0.0)
    acc_ref[...] += jnp.dot(xm, w1, preferred_element_type=jnp.float32,
                            precision=jax.lax.Precision.DEFAULT)

    @pl.when(i == KT - 1)
    def _():
        s = 1.0 / (1.0 + EPS) ** 0.5
        h = jnp.maximum(acc_ref[...] + b1_ref[...], 0.0)
        h = g1_ref[...] * h * s + be1_ref[...]
        h = jnp.maximum(jnp.dot(h, w2_ref[...], preferred_element_type=jnp.float32)
                        + b2_ref[...], 0.0)
        h = g2_ref[...] * h * s + be2_ref[...]
        lg = jnp.dot(h, w3_ref[...], preferred_element_type=jnp.float32) + b3_ref[...]
        c = jax.lax.broadcasted_iota(jnp.int32, lg.shape, 1)
        neg = jnp.where(c < OUT, lg, -jnp.inf)
        m = jnp.max(neg, axis=1, keepdims=True)
        ex = jnp.where(c < OUT, jnp.exp(lg - m), 0.0)
        lse = m + jnp.log(jnp.sum(ex, axis=1, keepdims=True))
        out_ref[...] = lg - lse


def kernel(x, adj_logits, W1, b1, gamma1, beta1, W2, b2, gamma2, beta2, W3, b3):
    adj = adj_logits.reshape(B, FLAT)

    thr = _sc_thresh(adj)

    w3p = jnp.pad(W3, ((0, 0), (0, 128 - OUT)))
    b3p = jnp.pad(b3, (0, 128 - OUT)).reshape(1, 128)

    out = pl.pallas_call(
        _mlp_body,
        grid=(KT,),
        in_specs=[
            pl.BlockSpec((B, KBLK), lambda i: (0, i)),
            pl.BlockSpec((B, KBLK), lambda i: (0, i)),
            pl.BlockSpec((B, L), lambda i: (0, 0)),
            pl.BlockSpec((KBLK, H1), lambda i: (i, 0)),
            pl.BlockSpec((1, H1), lambda i: (0, 0)),
            pl.BlockSpec((1, H1), lambda i: (0, 0)),
            pl.BlockSpec((1, H1), lambda i: (0, 0)),
            pl.BlockSpec((H1, H2), lambda i: (0, 0)),
            pl.BlockSpec((1, H2), lambda i: (0, 0)),
            pl.BlockSpec((1, H2), lambda i: (0, 0)),
            pl.BlockSpec((1, H2), lambda i: (0, 0)),
            pl.BlockSpec((H2, 128), lambda i: (0, 0)),
            pl.BlockSpec((1, 128), lambda i: (0, 0)),
        ],
        out_specs=pl.BlockSpec((B, 128), lambda i: (0, 0)),
        out_shape=jax.ShapeDtypeStruct((B, 128), jnp.float32),
        scratch_shapes=[pltpu.VMEM((B, H1), jnp.float32)],
    )(x, adj, thr, W1, b1.reshape(1, H1), gamma1.reshape(1, H1),
      beta1.reshape(1, H1), W2, b2.reshape(1, H2), gamma2.reshape(1, H2),
      beta2.reshape(1, H2), w3p, b3p)

    return out[:, :OUT]


# speedup vs baseline: 1.1437x; 1.1109x over previous
import functools

import jax
import jax.numpy as jnp
from jax import lax
from jax.experimental import pallas as pl
from jax.experimental.pallas import tpu as pltpu
from jax.experimental.pallas import tpu_sc as plsc

B = 100
N = 200
FLAT = N * N
NUM_EL = int(0.3 * N * N)
H1 = 512
H2 = 1024
OUT = 2
EPS = 1e-5

KBLK = 2048
KT = (FLAT + KBLK - 1) // KBLK

NC = 2
NS = 16
L = 16
NW = NC * NS
ROUNDS = (B + NW - 1) // NW
NBINS = 256
HIST_W = NBINS * L


def _sc_thresh_body(adj_ref, thr_ref, buf0, buf1, hist, stage, sem0, sem1):
    wid = lax.axis_index("s") * NC + lax.axis_index("c")
    lane = lax.iota(jnp.int32, L)
    lane_base = lane * NBINS
    ones = jnp.ones((L,), jnp.int32)
    bufs = (buf0, buf1)
    sems = (sem0, sem1)

    def start_fetch(r):
        g = r * NW + wid

        @pl.when(g < B)
        def _():
            pltpu.make_async_copy(adj_ref.at[g], bufs[r % 2], sems[r % 2]).start()

    start_fetch(0)
    for r in range(ROUNDS):
        g = r * NW + wid
        buf = bufs[r % 2]

        @pl.when(g < B)
        def _():
            pltpu.make_async_copy(adj_ref.at[g], buf, sems[r % 2]).wait()

        if r + 1 < ROUNDS:
            start_fetch(r + 1)

        @pl.when(g < B)
        def _():

            def run_pass(shift, prefix, target, first):
                @plsc.parallel_loop(0, HIST_W // L, unroll=8)
                def _(i):
                    hist[pl.ds(i * L, L)] = jnp.zeros((L,), jnp.int32)

                @plsc.parallel_loop(0, FLAT // L, unroll=16)
                def _(i):
                    v = buf[pl.ds(i * L, L)]
                    bits = lax.bitcast_convert_type(jnp.abs(v), jnp.int32)
                    digit = lax.shift_right_logical(bits, shift) & 0xFF
                    idx = digit + lane_base
                    if first:
                        plsc.addupdate_scatter(hist, [idx], ones)
                    else:
                        m = lax.shift_right_logical(bits, shift + 8) == prefix
                        plsc.addupdate_scatter(hist, [idx], ones, mask=m)

                def srch(j, carry):
                    cnt, d_star, excl = carry

                    def mrg(l, a):
                        return a + hist[pl.ds(l * NBINS + j * L, L)]

                    v = lax.fori_loop(0, L, mrg, jnp.zeros((L,), jnp.int32))
                    cs = plsc.cumsum(v)
                    tot = jnp.sum(v)
                    cross = (cnt <= target) & (cnt + tot > target)
                    inb = (cnt + cs) <= target
                    d_loc = jnp.sum(inb.astype(jnp.int32))
                    excl_loc = cnt + jnp.max(jnp.where(inb, cs, 0))
                    d_star = jnp.where(cross, j * L + d_loc, d_star)
                    excl = jnp.where(cross, excl_loc, excl)
                    return (cnt + tot, d_star, excl)

                _, d_star, excl = lax.fori_loop(
                    0, NBINS // L, srch,
                    (jnp.int32(0), jnp.int32(0), jnp.int32(0)))
                return (prefix << 8) | d_star, target - excl

            target = jnp.int32(FLAT - NUM_EL)
            prefix = jnp.int32(0)
            prefix, target = run_pass(23, prefix, target, True)
            prefix, target = run_pass(15, prefix, target, False)

            thr_bits = prefix << 15
            thr = lax.bitcast_convert_type(
                jnp.broadcast_to(thr_bits, (L,)), jnp.float32)
            stage[...] = thr
            pltpu.sync_copy(stage, thr_ref.at[g])


_sc_thresh = functools.partial(
    pl.kernel,
    out_type=jax.ShapeDtypeStruct((B, L), jnp.float32),
    mesh=plsc.VectorSubcoreMesh(core_axis_name="c", subcore_axis_name="s",
                                num_cores=NC, num_subcores=NS),
    compiler_params=pltpu.CompilerParams(needs_layout_passes=False),
    scratch_types=[
        pltpu.VMEM((FLAT,), jnp.float32),
        pltpu.VMEM((FLAT,), jnp.float32),
        pltpu.VMEM((HIST_W,), jnp.int32),
        pltpu.VMEM((L,), jnp.float32),
        pltpu.SemaphoreType.DMA,
        pltpu.SemaphoreType.DMA,
    ],
)(_sc_thresh_body)


def _mlp_body(x_ref, adj_ref, thr_ref, w1_ref, b1_ref, g1_ref, be1_ref,
              w2_ref, b2_ref, g2_ref, be2_ref, w3_ref, b3_ref,
              out_ref, acc_ref):
    i = pl.program_id(0)

    @pl.when(i == 0)
    def _():
        acc_ref[...] = jnp.zeros_like(acc_ref)

    thr = thr_ref[:, 0:1]
    col = i * KBLK + jax.lax.broadcasted_iota(jnp.int32, (B, KBLK), 1)
    keep = (col < FLAT) & (jnp.abs(adj_ref[...]) >= thr)
    xm = jnp.where(keep, x_ref[...], 0.0)
    row = i * KBLK + jax.lax.broadcasted_iota(jnp.int32, (KBLK, H1), 0)
    w1 = jnp.where(row < FLAT, w1_ref[...], 0.0)
    acc_ref[...] += jnp.dot(xm, w1, preferred_element_type=jnp.float32,
                            precision=jax.lax.Precision.DEFAULT)

    @pl.when(i == KT - 1)
    def _():
        s = 1.0 / (1.0 + EPS) ** 0.5
        h = jnp.maximum(acc_ref[...] + b1_ref[...], 0.0)
        h = g1_ref[...] * h * s + be1_ref[...]
        h = jnp.maximum(jnp.dot(h, w2_ref[...], preferred_element_type=jnp.float32)
                        + b2_ref[...], 0.0)
        h = g2_ref[...] * h * s + be2_ref[...]
        lg = jnp.dot(h, w3_ref[...], preferred_element_type=jnp.float32) + b3_ref[...]
        c = jax.lax.broadcasted_iota(jnp.int32, lg.shape, 1)
        neg = jnp.where(c < OUT, lg, -jnp.inf)
        m = jnp.max(neg, axis=1, keepdims=True)
        ex = jnp.where(c < OUT, jnp.exp(lg - m), 0.0)
        lse = m + jnp.log(jnp.sum(ex, axis=1, keepdims=True))
        out_ref[...] = lg - lse


def kernel(x, adj_logits, W1, b1, gamma1, beta1, W2, b2, gamma2, beta2, W3, b3):
    adj = adj_logits.reshape(B, FLAT)

    thr = _sc_thresh(adj)

    w3p = jnp.pad(W3, ((0, 0), (0, 128 - OUT)))
    b3p = jnp.pad(b3, (0, 128 - OUT)).reshape(1, 128)

    out = pl.pallas_call(
        _mlp_body,
        grid=(KT,),
        in_specs=[
            pl.BlockSpec((B, KBLK), lambda i: (0, i)),
            pl.BlockSpec((B, KBLK), lambda i: (0, i)),
            pl.BlockSpec((B, L), lambda i: (0, 0)),
            pl.BlockSpec((KBLK, H1), lambda i: (i, 0)),
            pl.BlockSpec((1, H1), lambda i: (0, 0)),
            pl.BlockSpec((1, H1), lambda i: (0, 0)),
            pl.BlockSpec((1, H1), lambda i: (0, 0)),
            pl.BlockSpec((H1, H2), lambda i: (0, 0)),
            pl.BlockSpec((1, H2), lambda i: (0, 0)),
            pl.BlockSpec((1, H2), lambda i: (0, 0)),
            pl.BlockSpec((1, H2), lambda i: (0, 0)),
            pl.BlockSpec((H2, 128), lambda i: (0, 0)),
            pl.BlockSpec((1, 128), lambda i: (0, 0)),
        ],
        out_specs=pl.BlockSpec((B, 128), lambda i: (0, 0)),
        out_shape=jax.ShapeDtypeStruct((B, 128), jnp.float32),
        scratch_shapes=[pltpu.VMEM((B, H1), jnp.float32)],
    )(x, adj, thr, W1, b1.reshape(1, H1), gamma1.reshape(1, H1),
      beta1.reshape(1, H1), W2, b2.reshape(1, H2), gamma2.reshape(1, H2),
      beta2.reshape(1, H2), w3p, b3p)

    return out[:, :OUT]


# speedup vs baseline: 1.2841x; 1.1228x over previous
import jax
import jax.numpy as jnp
from jax import lax
from jax.experimental import pallas as pl
from jax.experimental.pallas import tpu as pltpu

B = 100
N = 200
FLAT = N * N
NUM_EL = int(0.3 * N * N)
H1 = 512
H2 = 1024
OUT = 2
EPS = 1e-5

TKB = 2048
NT = (FLAT + TKB - 1) // TKB
LASTW = FLAT - (NT - 1) * TKB
BITS = 24


def _fused_body(adj_ref, x_any, w1_any, b1_ref, g1_ref, be1_ref,
                w2_ref, b2_ref, g2_ref, be2_ref, w3_ref, b3_ref,
                out_ref, xbuf, ring, acc_ref, xsem, rsem0, rsem1):
    rsems = (rsem0, rsem1)

    def ring_copy(t):
        w = LASTW if t == NT - 1 else TKB
        return pltpu.make_async_copy(
            w1_any.at[pl.ds(t * TKB, w), :],
            ring.at[t % 2, pl.ds(0, w), :],
            rsems[t % 2])

    pltpu.make_async_copy(x_any, xbuf, xsem).start()
    ring_copy(0).start()
    ring_copy(1).start()

    def bit_step(i, t):
        cand = t | jnp.left_shift(1, 30 - i)
        candf = lax.bitcast_convert_type(cand, jnp.float32)
        cnt = jnp.sum((jnp.abs(adj_ref[...]) >= candf).astype(jnp.int32),
                      axis=1, keepdims=True)
        return jnp.where(cnt >= NUM_EL, cand, t)

    t = lax.fori_loop(0, BITS, bit_step, jnp.zeros((B, 1), jnp.int32))
    thr = lax.bitcast_convert_type(t, jnp.float32)

    pltpu.make_async_copy(x_any, xbuf, xsem).wait()
    acc_ref[...] = jnp.zeros_like(acc_ref)

    for tt in range(NT):
        w = LASTW if tt == NT - 1 else TKB
        ring_copy(tt).wait()
        xm = jnp.where(
            jnp.abs(adj_ref[:, pl.ds(tt * TKB, w)]) >= thr,
            xbuf[:, pl.ds(tt * TKB, w)], 0.0)
        w1t = ring[tt % 2, pl.ds(0, w), :]
        acc_ref[...] += jnp.dot(xm, w1t, preferred_element_type=jnp.float32)
        if tt + 2 < NT:
            ring_copy(tt + 2).start()

    s = 1.0 / (1.0 + EPS) ** 0.5
    h = jnp.maximum(acc_ref[...] + b1_ref[...], 0.0)
    h = g1_ref[...] * h * s + be1_ref[...]
    h = jnp.maximum(jnp.dot(h, w2_ref[...], preferred_element_type=jnp.float32)
                    + b2_ref[...], 0.0)
    h = g2_ref[...] * h * s + be2_ref[...]
    lg = jnp.dot(h, w3_ref[...], preferred_element_type=jnp.float32) + b3_ref[...]
    c = jax.lax.broadcasted_iota(jnp.int32, lg.shape, 1)
    neg = jnp.where(c < OUT, lg, -jnp.inf)
    m = jnp.max(neg, axis=1, keepdims=True)
    ex = jnp.where(c < OUT, jnp.exp(lg - m), 0.0)
    lse = m + jnp.log(jnp.sum(ex, axis=1, keepdims=True))
    out_ref[...] = lg - lse


def kernel(x, adj_logits, W1, b1, gamma1, beta1, W2, b2, gamma2, beta2, W3, b3):
    adj = adj_logits.reshape(B, FLAT)

    w3p = jnp.pad(W3, ((0, 0), (0, 128 - OUT)))
    b3p = jnp.pad(b3, (0, 128 - OUT)).reshape(1, 128)

    out = pl.pallas_call(
        _fused_body,
        in_specs=[
            pl.BlockSpec((B, FLAT), lambda: (0, 0)),
            pl.BlockSpec(memory_space=pl.ANY),
            pl.BlockSpec(memory_space=pl.ANY),
            pl.BlockSpec((1, H1), lambda: (0, 0)),
            pl.BlockSpec((1, H1), lambda: (0, 0)),
            pl.BlockSpec((1, H1), lambda: (0, 0)),
            pl.BlockSpec((H1, H2), lambda: (0, 0)),
            pl.BlockSpec((1, H2), lambda: (0, 0)),
            pl.BlockSpec((1, H2), lambda: (0, 0)),
            pl.BlockSpec((1, H2), lambda: (0, 0)),
            pl.BlockSpec((H2, 128), lambda: (0, 0)),
            pl.BlockSpec((1, 128), lambda: (0, 0)),
        ],
        out_specs=pl.BlockSpec((B, 128), lambda: (0, 0)),
        out_shape=jax.ShapeDtypeStruct((B, 128), jnp.float32),
        scratch_shapes=[
            pltpu.VMEM((B, FLAT), jnp.float32),
            pltpu.VMEM((2, TKB, H1), jnp.float32),
            pltpu.VMEM((B, H1), jnp.float32),
            pltpu.SemaphoreType.DMA,
            pltpu.SemaphoreType.DMA,
            pltpu.SemaphoreType.DMA,
        ],
    )(adj, x, W1, b1.reshape(1, H1), gamma1.reshape(1, H1),
      beta1.reshape(1, H1), W2, b2.reshape(1, H2), gamma2.reshape(1, H2),
      beta2.reshape(1, H2), w3p, b3p)

    return out[:, :OUT]


# speedup vs baseline: 1.3799x; 1.0746x over previous
import jax
import jax.numpy as jnp
from jax import lax
from jax.experimental import pallas as pl
from jax.experimental.pallas import tpu as pltpu

B = 100
N = 200
FLAT = N * N
NUM_EL = int(0.3 * N * N)
H1 = 512
H2 = 1024
OUT = 2
EPS = 1e-5

TKB = 2048
NT = (FLAT + TKB - 1) // TKB
LASTW = FLAT - (NT - 1) * TKB
BITS = 24
RING = 6


def _fused_body(adj_any, x_any, w1_any, b1_ref, g1_ref, be1_ref,
                w2_ref, b2_ref, g2_ref, be2_ref, w3_ref, b3_ref,
                out_ref, keys, xbuf, ring, aring, alast, acc_ref,
                xsem, asem0, asem1, *rsems):

    def tile_w(t):
        return LASTW if t == NT - 1 else TKB

    asems = (asem0, asem1)

    def adj_copy(t):
        if t == NT - 1:
            return pltpu.make_async_copy(
                adj_any.at[:, pl.ds(t * TKB, LASTW)], alast, asems[t % 2])
        return pltpu.make_async_copy(
            adj_any.at[:, pl.ds(t * TKB, TKB)], aring.at[t % 2], asems[t % 2])

    def ring_copy(t):
        return pltpu.make_async_copy(
            w1_any.at[pl.ds(t * TKB, tile_w(t)), :],
            ring.at[t % RING, pl.ds(0, tile_w(t)), :],
            rsems[t % RING])

    adj_copy(0).start()
    adj_copy(1).start()
    for t in range(NT):
        w = tile_w(t)
        adj_copy(t).wait()
        src = alast[...] if t == NT - 1 else aring[t % 2]
        keys[:, pl.ds(t * TKB, w)] = lax.bitcast_convert_type(
            jnp.abs(src), jnp.int32)
        if t + 2 < NT:
            adj_copy(t + 2).start()

    pltpu.make_async_copy(x_any, xbuf, xsem).start()
    for t in range(RING):
        ring_copy(t).start()

    def bit_step(i, t):
        cand = t | jnp.left_shift(1, 30 - i)
        cnt = jnp.sum((keys[...] >= cand).astype(jnp.int32),
                      axis=1, keepdims=True)
        return jnp.where(cnt >= NUM_EL, cand, t)

    thr = lax.fori_loop(0, BITS, bit_step, jnp.zeros((B, 1), jnp.int32))

    pltpu.make_async_copy(x_any, xbuf, xsem).wait()
    acc_ref[...] = jnp.zeros_like(acc_ref)

    for tt in range(NT):
        w = tile_w(tt)
        ring_copy(tt).wait()
        xm = jnp.where(keys[:, pl.ds(tt * TKB, w)] >= thr,
                       xbuf[:, pl.ds(tt * TKB, w)], 0.0)
        w1t = ring[tt % RING, pl.ds(0, w), :]
        acc_ref[...] += jnp.dot(xm, w1t, preferred_element_type=jnp.float32)
        if tt + RING < NT:
            ring_copy(tt + RING).start()

    s = 1.0 / (1.0 + EPS) ** 0.5
    h = jnp.maximum(acc_ref[...] + b1_ref[...], 0.0)
    h = g1_ref[...] * h * s + be1_ref[...]
    h = jnp.maximum(jnp.dot(h, w2_ref[...], preferred_element_type=jnp.float32)
                    + b2_ref[...], 0.0)
    h = g2_ref[...] * h * s + be2_ref[...]
    lg = jnp.dot(h, w3_ref[...], preferred_element_type=jnp.float32) + b3_ref[...]
    c = jax.lax.broadcasted_iota(jnp.int32, lg.shape, 1)
    neg = jnp.where(c < OUT, lg, -jnp.inf)
    m = jnp.max(neg, axis=1, keepdims=True)
    ex = jnp.where(c < OUT, jnp.exp(lg - m), 0.0)
    lse = m + jnp.log(jnp.sum(ex, axis=1, keepdims=True))
    out_ref[...] = lg - lse


def kernel(x, adj_logits, W1, b1, gamma1, beta1, W2, b2, gamma2, beta2, W3, b3):
    adj = adj_logits.reshape(B, FLAT)

    w3p = jnp.pad(W3, ((0, 0), (0, 128 - OUT)))
    b3p = jnp.pad(b3, (0, 128 - OUT)).reshape(1, 128)

    out = pl.pallas_call(
        _fused_body,
        in_specs=[
            pl.BlockSpec(memory_space=pl.ANY),
            pl.BlockSpec(memory_space=pl.ANY),
            pl.BlockSpec(memory_space=pl.ANY),
            pl.BlockSpec((1, H1), lambda: (0, 0)),
            pl.BlockSpec((1, H1), lambda: (0, 0)),
            pl.BlockSpec((1, H1), lambda: (0, 0)),
            pl.BlockSpec((H1, H2), lambda: (0, 0)),
            pl.BlockSpec((1, H2), lambda: (0, 0)),
            pl.BlockSpec((1, H2), lambda: (0, 0)),
            pl.BlockSpec((1, H2), lambda: (0, 0)),
            pl.BlockSpec((H2, 128), lambda: (0, 0)),
            pl.BlockSpec((1, 128), lambda: (0, 0)),
        ],
        out_specs=pl.BlockSpec((B, 128), lambda: (0, 0)),
        out_shape=jax.ShapeDtypeStruct((B, 128), jnp.float32),
        scratch_shapes=[
            pltpu.VMEM((B, FLAT), jnp.int32),
            pltpu.VMEM((B, FLAT), jnp.float32),
            pltpu.VMEM((RING, TKB, H1), jnp.float32),
            pltpu.VMEM((2, B, TKB), jnp.float32),
            pltpu.VMEM((B, LASTW), jnp.float32),
            pltpu.VMEM((B, H1), jnp.float32),
            pltpu.SemaphoreType.DMA,
            pltpu.SemaphoreType.DMA,
            pltpu.SemaphoreType.DMA,
        ] + [pltpu.SemaphoreType.DMA] * RING,
        compiler_params=pltpu.CompilerParams(
            vmem_limit_bytes=63 * 1024 * 1024),
    )(adj, x, W1, b1.reshape(1, H1), gamma1.reshape(1, H1),
      beta1.reshape(1, H1), W2, b2.reshape(1, H2), gamma2.reshape(1, H2),
      beta2.reshape(1, H2), w3p, b3p)

    return out[:, :OUT]
